# baseline (speedup 1.0000x reference)
import jax
import jax.numpy as jnp
from jax import lax
from jax.experimental import pallas as pl
from jax.experimental.pallas import tpu as pltpu
from jax.experimental.pallas import tpu_sc as plsc

VOCAB = 1000000
BATCH = 16384
HIST = 50
CLASSES = 5
SCALE = 1.0 / 16384

NUM_CORES = 2
NUM_SUBCORES = 16
LANES = 16
NW = NUM_CORES * NUM_SUBCORES
B_PER_W = BATCH // NW
CHUNK = 8
N_CHUNKS = B_PER_W // CHUNK
HALF_CHUNKS = N_CHUNKS // 2
IDX_PER_CHUNK = CHUNK * HIST
IDX_HALF = HALF_CHUNKS * IDX_PER_CHUNK
SW = 16
IDX_ROWS = IDX_PER_CHUNK // SW
PADC = 8
WIDE = 128


def _sc_body(idx_hbm, zeros_hbm, table_hbm, sums_hbm,
             idx_v, rows_v, comp_v, dst_idx_v, acc_sh, bounce_sh,
             gsem0, gsem1, asem0, asem1):
    core = lax.axis_index("c")
    sub = lax.axis_index("s")
    wid = sub * NUM_CORES + core
    lane = lax.iota(jnp.int32, LANES)
    idx_row0 = wid * (B_PER_W * HIST // 128)
    my_acc = acc_sh.at[pl.ds(sub * B_PER_W, B_PER_W)]
    my_bounce = bounce_sh.at[sub]

    def fire_gathers(k, p, sem):
        off = jnp.bitwise_and(k, HALF_CHUNKS - 1) * IDX_PER_CHUNK
        for j in range(IDX_ROWS):
            o = off + j * SW
            pltpu.async_copy(
                table_hbm.at[idx_v.at[
                    lax.shift_right_logical(o, 7),
                    pl.ds(pl.multiple_of(jnp.bitwise_and(o, 127), SW), SW)]],
                rows_v.at[p, pl.ds(j * SW, SW)],
                sem,
            )

    def drain_gathers(k, p, sem):
        off = jnp.bitwise_and(k, HALF_CHUNKS - 1) * IDX_PER_CHUNK
        for j in range(IDX_ROWS):
            o = off + j * SW
            pltpu.make_async_copy(
                table_hbm.at[idx_v.at[
                    lax.shift_right_logical(o, 7),
                    pl.ds(pl.multiple_of(jnp.bitwise_and(o, 127), SW), SW)]],
                rows_v.at[p, pl.ds(j * SW, SW)],
                sem,
            ).wait()

    def fire_adds(k, p, sem):
        row0 = sub * B_PER_W + k * CHUNK
        for j in range(IDX_ROWS):
            q0, rem = divmod(j * SW, HIST)
            step = jnp.where(lane >= (HIST - rem), 1, 0)
            dst_idx_v[j, pl.ds(0, LANES)] = row0 + q0 + step
        for j in range(IDX_ROWS):
            pltpu.async_copy(
                comp_v.at[p, pl.ds(j * SW, SW)],
                acc_sh.at[dst_idx_v.at[j]],
                sem,
                add=True,
            )

    def drain_adds(p, sem):
        for j in range(IDX_ROWS):
            pltpu.make_async_copy(
                comp_v.at[p, pl.ds(j * SW, SW)],
                acc_sh.at[dst_idx_v.at[j]],
                sem,
            ).wait()

    def chunk_step(k, p, sem_g_mine, sem_g_other, sem_a_mine, sem_a_other):
        @pl.when(k == HALF_CHUNKS - 1)
        def _():
            pltpu.sync_copy(
                idx_hbm.at[pl.ds(idx_row0 + IDX_HALF // 128,
                                 IDX_HALF // 128)], idx_v)

        @pl.when(k + 1 < N_CHUNKS)
        def _():
            fire_gathers(k + 1, 1 - p, sem_g_other)

        drain_gathers(k, p, sem_g_mine)

        pltpu.sync_copy(rows_v.at[p, :, pl.ds(0, PADC)], my_bounce)
        pltpu.sync_copy(my_bounce, comp_v.at[p])

        @pl.when(k > 0)
        def _():
            drain_adds(1 - p, sem_a_other)

        fire_adds(k, p, sem_a_mine)

    pltpu.sync_copy(idx_hbm.at[pl.ds(idx_row0, IDX_HALF // 128)], idx_v)
    pltpu.sync_copy(zeros_hbm, my_acc)
    fire_gathers(0, 0, gsem0)

    def loop_body(k, _):
        parity = jnp.bitwise_and(k, 1)

        @pl.when(parity == 0)
        def _():
            chunk_step(k, 0, gsem0, gsem1, asem0, asem1)

        @pl.when(parity == 1)
        def _():
            chunk_step(k, 1, gsem1, gsem0, asem1, asem0)

        return ()

    lax.fori_loop(0, N_CHUNKS, loop_body, ())

    drain_adds(1, asem1)
    pltpu.sync_copy(my_acc, sums_hbm.at[wid])


@jax.jit
def _embed_sums(idx1d, zeros, table):
    mesh = plsc.VectorSubcoreMesh(
        core_axis_name="c", subcore_axis_name="s",
        num_cores=NUM_CORES, num_subcores=NUM_SUBCORES)
    return pl.kernel(
        _sc_body,
        out_type=jax.ShapeDtypeStruct(
            (NW, B_PER_W, PADC), jnp.float32),
        mesh=mesh,
        compiler_params=pltpu.CompilerParams(use_tc_tiling_on_sc=False),
        scratch_types=[
            pltpu.VMEM((IDX_HALF // 128, 128), jnp.int32),
            pltpu.VMEM((2, IDX_PER_CHUNK, WIDE), jnp.float32),
            pltpu.VMEM((2, IDX_PER_CHUNK, PADC), jnp.float32),
            pltpu.VMEM((IDX_ROWS, SW), jnp.int32),
            pltpu.VMEM_SHARED(
                (NUM_SUBCORES * B_PER_W, PADC), jnp.float32),
            pltpu.VMEM_SHARED(
                (NUM_SUBCORES, IDX_PER_CHUNK, PADC), jnp.float32),
            pltpu.SemaphoreType.DMA,
            pltpu.SemaphoreType.DMA,
            pltpu.SemaphoreType.DMA,
            pltpu.SemaphoreType.DMA,
        ],
    )(idx1d, zeros, table)


def _softmax_body(s_ref, o_ref):
    s = s_ref[:, :CLASSES] * SCALE
    m = jnp.max(s, axis=-1, keepdims=True)
    e = jnp.exp(s - m)
    o_ref[...] = e / jnp.sum(e, axis=-1, keepdims=True)


@jax.jit
def _softmax(sums):
    return pl.pallas_call(
        _softmax_body,
        out_shape=jax.ShapeDtypeStruct((BATCH, CLASSES), jnp.float32),
        grid=(8,),
        in_specs=[pl.BlockSpec((BATCH // 8, PADC), lambda i: (i, 0))],
        out_specs=pl.BlockSpec((BATCH // 8, CLASSES), lambda i: (i, 0)),
    )(sums)


def kernel(indices, table):
    idx1d = jnp.pad(jnp.bitwise_and(
        indices.astype(jnp.int32), jnp.int32(0xFFFFF)).reshape(
        BATCH * HIST // 128, 128), ((0, 8), (0, 0)))
    zeros = jnp.zeros((B_PER_W, PADC), jnp.float32)
    tablew = jnp.pad(table, ((0, 0), (0, WIDE - CLASSES)))
    sums = _embed_sums(idx1d, zeros, tablew).reshape(BATCH, PADC)
    return _softmax(sums)


# speedup vs baseline: 1.0056x; 1.0056x over previous
import jax
import jax.numpy as jnp
from jax import lax
from jax.experimental import pallas as pl
from jax.experimental.pallas import tpu as pltpu
from jax.experimental.pallas import tpu_sc as plsc

VOCAB = 1000000
BATCH = 16384
HIST = 50
CLASSES = 5
SCALE = 1.0 / 16384

NUM_CORES = 2
NUM_SUBCORES = 16
LANES = 16
NW = NUM_CORES * NUM_SUBCORES
B_PER_W = BATCH // NW
CHUNK = 8
N_CHUNKS = B_PER_W // CHUNK
HALF_CHUNKS = N_CHUNKS // 2
IDX_PER_CHUNK = CHUNK * HIST
IDX_HALF = HALF_CHUNKS * IDX_PER_CHUNK
SW = 16
IDX_ROWS = IDX_PER_CHUNK // SW
PADC = 8
WIDE = 128


def _sc_body(idx_hbm, zeros_hbm, table_hbm, sums_hbm,
             idx_v, rows_v, comp_v, dst_idx_v, acc_sh, bounce_sh,
             gsem0, gsem1, asem0, asem1):
    core = lax.axis_index("c")
    sub = lax.axis_index("s")
    wid = sub * NUM_CORES + core
    lane = lax.iota(jnp.int32, LANES)
    idx_row0 = wid * (B_PER_W * HIST // 128)
    my_acc = acc_sh.at[pl.ds(sub * B_PER_W, B_PER_W)]
    my_bounce = bounce_sh.at[sub]

    def fire_gathers(k, p, sem):
        off = jnp.bitwise_and(k, HALF_CHUNKS - 1) * IDX_PER_CHUNK
        for j in range(IDX_ROWS):
            o = off + j * SW
            pltpu.async_copy(
                table_hbm.at[idx_v.at[
                    lax.shift_right_logical(o, 7),
                    pl.ds(pl.multiple_of(jnp.bitwise_and(o, 127), SW), SW)]],
                rows_v.at[p, pl.ds(j * SW, SW)],
                sem,
            )

    def drain_gathers(k, p, sem):
        pltpu.make_async_copy(
            table_hbm.at[pl.ds(0, IDX_PER_CHUNK)],
            rows_v.at[p],
            sem,
        ).wait()

    def fire_adds(k, p, sem):
        row0 = sub * B_PER_W + k * CHUNK
        for j in range(IDX_ROWS):
            q0, rem = divmod(j * SW, HIST)
            step = jnp.where(lane >= (HIST - rem), 1, 0)
            dst_idx_v[j, pl.ds(0, LANES)] = row0 + q0 + step
        for j in range(IDX_ROWS):
            pltpu.async_copy(
                comp_v.at[p, pl.ds(j * SW, SW)],
                acc_sh.at[dst_idx_v.at[j]],
                sem,
                add=True,
            )

    def drain_adds(p, sem):
        pltpu.make_async_copy(
            zeros_hbm.at[pl.ds(0, IDX_PER_CHUNK)],
            comp_v.at[p],
            sem,
        ).wait()

    def chunk_step(k, p, sem_g_mine, sem_g_other, sem_a_mine, sem_a_other):
        @pl.when(k == HALF_CHUNKS - 1)
        def _():
            pltpu.sync_copy(
                idx_hbm.at[pl.ds(idx_row0 + IDX_HALF // 128,
                                 IDX_HALF // 128)], idx_v)

        @pl.when(k + 1 < N_CHUNKS)
        def _():
            fire_gathers(k + 1, 1 - p, sem_g_other)

        drain_gathers(k, p, sem_g_mine)

        pltpu.sync_copy(rows_v.at[p, :, pl.ds(0, PADC)], my_bounce)
        pltpu.sync_copy(my_bounce, comp_v.at[p])

        @pl.when(k > 0)
        def _():
            drain_adds(1 - p, sem_a_other)

        fire_adds(k, p, sem_a_mine)

    pltpu.sync_copy(idx_hbm.at[pl.ds(idx_row0, IDX_HALF // 128)], idx_v)
    pltpu.sync_copy(zeros_hbm, my_acc)
    fire_gathers(0, 0, gsem0)

    def loop_body(k, _):
        parity = jnp.bitwise_and(k, 1)

        @pl.when(parity == 0)
        def _():
            chunk_step(k, 0, gsem0, gsem1, asem0, asem1)

        @pl.when(parity == 1)
        def _():
            chunk_step(k, 1, gsem1, gsem0, asem1, asem0)

        return ()

    lax.fori_loop(0, N_CHUNKS, loop_body, ())

    drain_adds(1, asem1)
    pltpu.sync_copy(my_acc, sums_hbm.at[wid])


@jax.jit
def _embed_sums(idx1d, zeros, table):
    mesh = plsc.VectorSubcoreMesh(
        core_axis_name="c", subcore_axis_name="s",
        num_cores=NUM_CORES, num_subcores=NUM_SUBCORES)
    return pl.kernel(
        _sc_body,
        out_type=jax.ShapeDtypeStruct(
            (NW, B_PER_W, PADC), jnp.float32),
        mesh=mesh,
        compiler_params=pltpu.CompilerParams(use_tc_tiling_on_sc=False),
        scratch_types=[
            pltpu.VMEM((IDX_HALF // 128, 128), jnp.int32),
            pltpu.VMEM((2, IDX_PER_CHUNK, WIDE), jnp.float32),
            pltpu.VMEM((2, IDX_PER_CHUNK, PADC), jnp.float32),
            pltpu.VMEM((IDX_ROWS, SW), jnp.int32),
            pltpu.VMEM_SHARED(
                (NUM_SUBCORES * B_PER_W, PADC), jnp.float32),
            pltpu.VMEM_SHARED(
                (NUM_SUBCORES, IDX_PER_CHUNK, PADC), jnp.float32),
            pltpu.SemaphoreType.DMA,
            pltpu.SemaphoreType.DMA,
            pltpu.SemaphoreType.DMA,
            pltpu.SemaphoreType.DMA,
        ],
    )(idx1d, zeros, table)


def _softmax_body(s_ref, o_ref):
    s = s_ref[:, :CLASSES] * SCALE
    m = jnp.max(s, axis=-1, keepdims=True)
    e = jnp.exp(s - m)
    o_ref[...] = e / jnp.sum(e, axis=-1, keepdims=True)


@jax.jit
def _softmax(sums):
    return pl.pallas_call(
        _softmax_body,
        out_shape=jax.ShapeDtypeStruct((BATCH, CLASSES), jnp.float32),
        grid=(8,),
        in_specs=[pl.BlockSpec((BATCH // 8, PADC), lambda i: (i, 0))],
        out_specs=pl.BlockSpec((BATCH // 8, CLASSES), lambda i: (i, 0)),
    )(sums)


def kernel(indices, table):
    idx1d = jnp.pad(jnp.bitwise_and(
        indices.astype(jnp.int32), jnp.int32(0xFFFFF)).reshape(
        BATCH * HIST // 128, 128), ((0, 8), (0, 0)))
    zeros = jnp.zeros((B_PER_W, PADC), jnp.float32)
    tablew = jnp.pad(table, ((0, 0), (0, WIDE - CLASSES)))
    sums = _embed_sums(idx1d, zeros, tablew).reshape(BATCH, PADC)
    return _softmax(sums)
